# baseline (speedup 1.0000x reference)
import functools

import jax
import jax.numpy as jnp
from jax import lax
from jax.experimental import pallas as pl
from jax.experimental.pallas import tpu as pltpu
from jax.experimental.pallas import tpu_sc as plsc

N_NODES = 10000
N_EDGES = 320000
D = 128

NC = 2
NS = 16
NW = NC * NS

K = 80
CHUNKS = 126
E_PER_W = K * CHUNKS
E_PAD = NW * E_PER_W
N_PAD = 10112
ROWS_PER_TILE = N_PAD // NS
LANES = D // 16
NBUF = 3


def _mm_body(x_ref, w_ref, o_ref):
    o_ref[...] = jnp.dot(x_ref[...], w_ref[...],
                         preferred_element_type=jnp.float32)


def _matmul(x, W):
    RB = 1000
    return pl.pallas_call(
        _mm_body,
        grid=(N_NODES // RB,),
        in_specs=[
            pl.BlockSpec((RB, D), lambda i: (i, 0)),
            pl.BlockSpec((D, D), lambda i: (0, 0)),
        ],
        out_specs=pl.BlockSpec((RB, D), lambda i: (i, 0)),
        out_shape=jax.ShapeDtypeStruct((N_NODES, D), jnp.float32),
    )(x, W)


_sc_mesh = plsc.VectorSubcoreMesh(core_axis_name="c", subcore_axis_name="s")

_BCAST_DNUMS = lax.GatherDimensionNumbers(
    offset_dims=(), collapsed_slice_dims=(0,), start_index_map=(0,))


def _lane_bcast(vec, lane):
    idx = jnp.full((16, 1), lane, jnp.int32)
    return lax.gather(vec, idx, _BCAST_DNUMS, (1,),
                      mode=lax.GatherScatterMode.PROMISE_IN_BOUNDS)


@functools.partial(
    pl.kernel,
    out_type=jax.ShapeDtypeStruct((2, N_PAD, D), jnp.float32),
    mesh=_sc_mesh,
    compiler_params=pltpu.CompilerParams(use_tc_tiling_on_sc=False),
    scratch_types=[
        [pltpu.VMEM((K,), jnp.int32) for _ in range(NBUF)],
        [pltpu.VMEM((K,), jnp.int32) for _ in range(NBUF)],
        [pltpu.VMEM((K,), jnp.float32) for _ in range(NBUF)],
        [pltpu.VMEM((K, D // 2), jnp.int32) for _ in range(NBUF)],
        [pltpu.VMEM((K, D), jnp.float32) for _ in range(NBUF)],
        pltpu.VMEM_SHARED((N_PAD, D), jnp.float32),
        [pltpu.SemaphoreType.DMA for _ in range(NBUF)],
        [pltpu.SemaphoreType.DMA for _ in range(NBUF)],
        [pltpu.SemaphoreType.DMA for _ in range(NBUF)],
    ],
)
def _sc_scatter(sup_hbm, rows_hbm, cols_hbm, vals_hbm, zeros_hbm,
                out_hbm,
                cols_b, rows_b, vals_b, gath_b, scal_b, acc_sh,
                tsem, gsem, ssem):
    c = lax.axis_index("c")
    s = lax.axis_index("s")
    wid = c * NS + s
    cbase = wid * CHUNKS

    rbase = s * ROWS_PER_TILE
    rslice = pl.ds(rbase, ROWS_PER_TILE)
    pltpu.sync_copy(zeros_hbm.at[rslice], acc_sh.at[rslice])
    plsc.subcore_barrier()

    def tstart(ci, b):
        off = (cbase + ci) * K
        pltpu.async_copy(cols_hbm.at[pl.ds(off, K)], cols_b[b], tsem[b])
        pltpu.async_copy(rows_hbm.at[pl.ds(off, K)], rows_b[b], tsem[b])
        pltpu.async_copy(vals_hbm.at[pl.ds(off, K)], vals_b[b], tsem[b])

    def twait(b):
        pltpu.make_async_copy(cols_hbm.at[pl.ds(0, K)], cols_b[b],
                              tsem[b]).wait()
        pltpu.make_async_copy(rows_hbm.at[pl.ds(0, K)], rows_b[b],
                              tsem[b]).wait()
        pltpu.make_async_copy(vals_hbm.at[pl.ds(0, K)], vals_b[b],
                              tsem[b]).wait()

    def gstart(b):
        pltpu.async_copy(sup_hbm.at[cols_b[b]], gath_b[b], gsem[b])

    def gwait(b):
        pltpu.make_async_copy(sup_hbm.at[cols_b[b]], gath_b[b],
                              gsem[b]).wait()

    def sstart(b):
        pltpu.async_copy(scal_b[b], acc_sh.at[rows_b[b]], ssem[b], add=True)

    def swait(b):
        pltpu.make_async_copy(scal_b[b], acc_sh.at[rows_b[b]],
                              ssem[b]).wait()

    def scale(b):
        g_ref = gath_b[b]
        o_ref = scal_b[b]
        v_ref = vals_b[b]

        @pl.loop(0, K // 16)
        def _grp(g):
            vgrp = v_ref[pl.ds(g * 16, 16)]
            for l in range(16):
                vv = _lane_bcast(vgrp, l)
                e = g * 16 + l
                for blk in range(D // 32):
                    w = g_ref[e, pl.ds(blk * 16, 16)]
                    va = lax.bitcast_convert_type(w << 16, jnp.float32)
                    vb = lax.bitcast_convert_type(
                        w & jnp.int32(-65536), jnp.float32)
                    o_ref[e, pl.ds(blk * 32, 16)] = va * vv
                    o_ref[e, pl.ds(blk * 32 + 16, 16)] = vb * vv

    tstart(0, 0)
    tstart(1, 1)
    twait(0)
    gstart(0)

    @pl.loop(0, CHUNKS, step=NBUF)
    def _iter(i):
        for db in range(NBUF):
            b = db
            b1 = (db + 1) % NBUF
            b2 = (db + 2) % NBUF
            ci = i + db

            gwait(b)
            scale(b)
            sstart(b)

            @pl.when(ci + 2 < CHUNKS)
            def _():
                @pl.when(ci >= 1)
                def _():
                    swait(b2)
                tstart(ci + 2, b2)

            @pl.when(ci + 1 < CHUNKS)
            def _():
                twait(b1)
                gstart(b1)

    swait((CHUNKS - 3) % NBUF)
    swait((CHUNKS - 2) % NBUF)
    swait((CHUNKS - 1) % NBUF)

    plsc.subcore_barrier()
    pltpu.sync_copy(acc_sh.at[rslice], out_hbm.at[c, rslice])


def _comb_body(p0_ref, p1_ref, b_ref, o_ref):
    o_ref[...] = p0_ref[0] + p1_ref[0] + b_ref[...]


def _combine(p, b):
    RB = 1000
    return pl.pallas_call(
        _comb_body,
        grid=(N_NODES // RB,),
        in_specs=[
            pl.BlockSpec((1, RB, D), lambda i: (0, i, 0)),
            pl.BlockSpec((1, RB, D), lambda i: (1, i, 0)),
            pl.BlockSpec((1, D), lambda i: (0, 0)),
        ],
        out_specs=pl.BlockSpec((RB, D), lambda i: (i, 0)),
        out_shape=jax.ShapeDtypeStruct((N_NODES, D), jnp.float32),
    )(p, p, b)


def kernel(x, rows, cols, values, W, b):
    support = _matmul(x, W)
    sup_perm = support.reshape(N_NODES, D // 32, 2, 16).transpose(
        0, 1, 3, 2).reshape(N_NODES, D)
    sup_bf16 = sup_perm.astype(jnp.bfloat16)
    sup_packed = lax.bitcast_convert_type(
        sup_bf16.reshape(N_NODES, D // 2, 2), jnp.int32)
    pad = E_PAD - N_EDGES
    rows1 = jnp.concatenate([rows, jnp.zeros((pad,), rows.dtype)])
    cols1 = jnp.concatenate([cols, jnp.zeros((pad,), cols.dtype)])
    vals1 = jnp.concatenate([values, jnp.zeros((pad,), values.dtype)])
    zeros = jnp.zeros((N_PAD, D), jnp.float32)
    p = _sc_scatter(sup_packed, rows1, cols1, vals1, zeros)
    return _combine(p, b)


# speedup vs baseline: 1.5262x; 1.5262x over previous
import functools

import jax
import jax.numpy as jnp
from jax import lax
from jax.experimental import pallas as pl
from jax.experimental.pallas import tpu as pltpu
from jax.experimental.pallas import tpu_sc as plsc

N_NODES = 10000
N_EDGES = 320000
D = 128

NC = 2
NS = 16
NW = NC * NS

K = 112
CHUNKS = 90
E_PER_W = K * CHUNKS
E_PAD = NW * E_PER_W
N_PAD = 10112
ROWS_PER_TILE = N_PAD // NS
LANES = D // 16
NBUF = 3


def _mm_body(x_ref, w_ref, o_ref):
    o_ref[...] = jnp.dot(x_ref[...], w_ref[...],
                         preferred_element_type=jnp.float32)


def _matmul(x, W):
    RB = 1000
    return pl.pallas_call(
        _mm_body,
        grid=(N_NODES // RB,),
        in_specs=[
            pl.BlockSpec((RB, D), lambda i: (i, 0)),
            pl.BlockSpec((D, D), lambda i: (0, 0)),
        ],
        out_specs=pl.BlockSpec((RB, D), lambda i: (i, 0)),
        out_shape=jax.ShapeDtypeStruct((N_NODES, D), jnp.float32),
    )(x, W)


_sc_mesh = plsc.VectorSubcoreMesh(core_axis_name="c", subcore_axis_name="s")

_BCAST_DNUMS = lax.GatherDimensionNumbers(
    offset_dims=(), collapsed_slice_dims=(0,), start_index_map=(0,))


def _lane_bcast(vec, lane):
    idx = jnp.full((16, 1), lane, jnp.int32)
    return lax.gather(vec, idx, _BCAST_DNUMS, (1,),
                      mode=lax.GatherScatterMode.PROMISE_IN_BOUNDS)


@functools.partial(
    pl.kernel,
    out_type=jax.ShapeDtypeStruct((2, N_PAD, D), jnp.float32),
    mesh=_sc_mesh,
    scratch_types=[
        [pltpu.VMEM((K,), jnp.int32) for _ in range(NBUF)],
        [pltpu.VMEM((K,), jnp.int32) for _ in range(NBUF)],
        [pltpu.VMEM((K,), jnp.float32) for _ in range(NBUF)],
        [pltpu.VMEM((K, D), jnp.float32) for _ in range(NBUF)],
        pltpu.VMEM_SHARED((N_PAD, D), jnp.float32),
        [pltpu.SemaphoreType.DMA for _ in range(NBUF)],
        [pltpu.SemaphoreType.DMA for _ in range(NBUF)],
        [pltpu.SemaphoreType.DMA for _ in range(NBUF)],
    ],
)
def _sc_scatter(sup_hbm, rows_hbm, cols_hbm, vals_hbm, zeros_hbm,
                out_hbm,
                cols_b, rows_b, vals_b, gath_b, acc_sh,
                tsem, gsem, ssem):
    c = lax.axis_index("c")
    s = lax.axis_index("s")
    wid = c * NS + s
    cbase = wid * CHUNKS

    rbase = s * ROWS_PER_TILE
    rslice = pl.ds(rbase, ROWS_PER_TILE)
    pltpu.sync_copy(zeros_hbm.at[rslice], acc_sh.at[rslice])
    plsc.subcore_barrier()

    def tstart(ci, b):
        off = (cbase + ci) * K
        pltpu.async_copy(cols_hbm.at[pl.ds(off, K)], cols_b[b], tsem[b])
        pltpu.async_copy(rows_hbm.at[pl.ds(off, K)], rows_b[b], tsem[b])
        pltpu.async_copy(vals_hbm.at[pl.ds(off, K)], vals_b[b], tsem[b])

    def twait(b):
        pltpu.make_async_copy(cols_hbm.at[pl.ds(0, K)], cols_b[b],
                              tsem[b]).wait()
        pltpu.make_async_copy(rows_hbm.at[pl.ds(0, K)], rows_b[b],
                              tsem[b]).wait()
        pltpu.make_async_copy(vals_hbm.at[pl.ds(0, K)], vals_b[b],
                              tsem[b]).wait()

    def gstart(b):
        pltpu.async_copy(sup_hbm.at[cols_b[b]], gath_b[b], gsem[b])

    def gwait(b):
        pltpu.make_async_copy(sup_hbm.at[cols_b[b]], gath_b[b],
                              gsem[b]).wait()

    def sstart(b):
        pltpu.async_copy(gath_b[b], acc_sh.at[rows_b[b]], ssem[b], add=True)

    def swait(b):
        pltpu.make_async_copy(gath_b[b], acc_sh.at[rows_b[b]],
                              ssem[b]).wait()

    def scale(b):
        g_ref = gath_b[b]
        v_ref = vals_b[b]

        @pl.loop(0, K // 16)
        def _grp(g):
            vgrp = v_ref[pl.ds(g * 16, 16)]
            for l in range(16):
                vv = _lane_bcast(vgrp, l)
                e = g * 16 + l
                for j in range(LANES):
                    sl = pl.ds(j * 16, 16)
                    g_ref[e, sl] = g_ref[e, sl] * vv

    tstart(0, 0)
    tstart(1, 1)
    twait(0)
    gstart(0)

    @pl.loop(0, CHUNKS, step=NBUF)
    def _iter(i):
        for db in range(NBUF):
            b = db
            b1 = (db + 1) % NBUF
            b2 = (db + 2) % NBUF
            ci = i + db

            gwait(b)

            @pl.when(ci + 1 < CHUNKS)
            def _():
                twait(b1)
                gstart(b1)

            scale(b)
            sstart(b)

            @pl.when(ci + 2 < CHUNKS)
            def _():
                @pl.when(ci >= 1)
                def _():
                    swait(b2)
                tstart(ci + 2, b2)

    swait((CHUNKS - 3) % NBUF)
    swait((CHUNKS - 2) % NBUF)
    swait((CHUNKS - 1) % NBUF)

    plsc.subcore_barrier()
    pltpu.sync_copy(acc_sh.at[rslice], out_hbm.at[c, rslice])


def _comb_body(p0_ref, p1_ref, b_ref, o_ref):
    o_ref[...] = p0_ref[0] + p1_ref[0] + b_ref[...]


def _combine(p, b):
    RB = 1000
    return pl.pallas_call(
        _comb_body,
        grid=(N_NODES // RB,),
        in_specs=[
            pl.BlockSpec((1, RB, D), lambda i: (0, i, 0)),
            pl.BlockSpec((1, RB, D), lambda i: (1, i, 0)),
            pl.BlockSpec((1, D), lambda i: (0, 0)),
        ],
        out_specs=pl.BlockSpec((RB, D), lambda i: (i, 0)),
        out_shape=jax.ShapeDtypeStruct((N_NODES, D), jnp.float32),
    )(p, p, b)


def kernel(x, rows, cols, values, W, b):
    support = _matmul(x, W)
    pad = E_PAD - N_EDGES
    rows1 = jnp.concatenate([rows, jnp.zeros((pad,), rows.dtype)])
    cols1 = jnp.concatenate([cols, jnp.zeros((pad,), cols.dtype)])
    vals1 = jnp.concatenate([values, jnp.zeros((pad,), values.dtype)])
    zeros = jnp.zeros((N_PAD, D), jnp.float32)
    p = _sc_scatter(support, rows1, cols1, vals1, zeros)
    return _combine(p, b)


# speedup vs baseline: 1.5768x; 1.0331x over previous
import functools

import jax
import jax.numpy as jnp
from jax import lax
from jax.experimental import pallas as pl
from jax.experimental.pallas import tpu as pltpu
from jax.experimental.pallas import tpu_sc as plsc

N_NODES = 10000
N_EDGES = 320000
D = 128

NC = 2
NS = 16
NW = NC * NS

K = 112
CHUNKS = 90
E_PER_W = K * CHUNKS
E_PAD = NW * E_PER_W
N_PAD = 10112
ROWS_PER_TILE = N_PAD // NS
LANES = D // 16
NBUF = 3


def _mm_body(x_ref, w_ref, o_ref):
    o_ref[...] = jnp.dot(x_ref[...], w_ref[...],
                         preferred_element_type=jnp.float32)


def _matmul(x, W):
    RB = 1000
    return pl.pallas_call(
        _mm_body,
        grid=(N_NODES // RB,),
        in_specs=[
            pl.BlockSpec((RB, D), lambda i: (i, 0)),
            pl.BlockSpec((D, D), lambda i: (0, 0)),
        ],
        out_specs=pl.BlockSpec((RB, D), lambda i: (i, 0)),
        out_shape=jax.ShapeDtypeStruct((N_NODES, D), jnp.float32),
    )(x, W)


_sc_mesh = plsc.VectorSubcoreMesh(core_axis_name="c", subcore_axis_name="s")

_BCAST_DNUMS = lax.GatherDimensionNumbers(
    offset_dims=(), collapsed_slice_dims=(0,), start_index_map=(0,))


def _lane_bcast(vec, lane):
    idx = jnp.full((16, 1), lane, jnp.int32)
    return lax.gather(vec, idx, _BCAST_DNUMS, (1,),
                      mode=lax.GatherScatterMode.PROMISE_IN_BOUNDS)


@functools.partial(
    pl.kernel,
    out_type=jax.ShapeDtypeStruct((2, N_PAD, D), jnp.float32),
    mesh=_sc_mesh,
    scratch_types=[
        [pltpu.VMEM((K,), jnp.int32) for _ in range(NBUF)],
        [pltpu.VMEM((K,), jnp.int32) for _ in range(NBUF)],
        [pltpu.VMEM((K,), jnp.float32) for _ in range(NBUF)],
        [pltpu.VMEM((K, D), jnp.float32) for _ in range(NBUF)],
        pltpu.VMEM_SHARED((N_PAD, D), jnp.float32),
        [pltpu.SemaphoreType.DMA for _ in range(NBUF)],
        [pltpu.SemaphoreType.DMA for _ in range(NBUF)],
        [pltpu.SemaphoreType.DMA for _ in range(NBUF)],
    ],
)
def _sc_scatter(sup_hbm, rows_hbm, cols_hbm, vals_hbm,
                out_hbm,
                cols_b, rows_b, vals_b, gath_b, acc_sh,
                tsem, gsem, ssem):
    c = lax.axis_index("c")
    s = lax.axis_index("s")
    wid = c * NS + s
    cbase = wid * CHUNKS

    rbase = s * ROWS_PER_TILE
    rslice = pl.ds(rbase, ROWS_PER_TILE)
    z_ref = gath_b[0]

    @pl.loop(0, K)
    def _zrow(e):
        for j in range(LANES):
            z_ref[e, pl.ds(j * 16, 16)] = jnp.zeros((16,), jnp.float32)

    for r in range(ROWS_PER_TILE // K):
        pltpu.sync_copy(z_ref, acc_sh.at[pl.ds(rbase + r * K, K)])
    rem = ROWS_PER_TILE % K
    pltpu.sync_copy(z_ref.at[pl.ds(0, rem)],
                    acc_sh.at[pl.ds(rbase + ROWS_PER_TILE - rem, rem)])
    plsc.subcore_barrier()

    def tstart(ci, b):
        off = (cbase + ci) * K
        pltpu.async_copy(cols_hbm.at[pl.ds(off, K)], cols_b[b], tsem[b])
        pltpu.async_copy(rows_hbm.at[pl.ds(off, K)], rows_b[b], tsem[b])
        pltpu.async_copy(vals_hbm.at[pl.ds(off, K)], vals_b[b], tsem[b])

    def twait(b):
        pltpu.make_async_copy(cols_hbm.at[pl.ds(0, K)], cols_b[b],
                              tsem[b]).wait()
        pltpu.make_async_copy(rows_hbm.at[pl.ds(0, K)], rows_b[b],
                              tsem[b]).wait()
        pltpu.make_async_copy(vals_hbm.at[pl.ds(0, K)], vals_b[b],
                              tsem[b]).wait()

    def gstart(b):
        pltpu.async_copy(sup_hbm.at[cols_b[b]], gath_b[b], gsem[b])

    def gwait(b):
        pltpu.make_async_copy(sup_hbm.at[cols_b[b]], gath_b[b],
                              gsem[b]).wait()

    def sstart(b):
        pltpu.async_copy(gath_b[b], acc_sh.at[rows_b[b]], ssem[b], add=True)

    def swait(b):
        pltpu.make_async_copy(gath_b[b], acc_sh.at[rows_b[b]],
                              ssem[b]).wait()

    def scale(b):
        g_ref = gath_b[b]
        v_ref = vals_b[b]

        @pl.loop(0, K // 16)
        def _grp(g):
            vgrp = v_ref[pl.ds(g * 16, 16)]
            for l in range(16):
                vv = _lane_bcast(vgrp, l)
                e = g * 16 + l
                for j in range(LANES):
                    sl = pl.ds(j * 16, 16)
                    g_ref[e, sl] = g_ref[e, sl] * vv

    tstart(0, 0)
    tstart(1, 1)
    twait(0)
    gstart(0)

    @pl.loop(0, CHUNKS, step=NBUF)
    def _iter(i):
        for db in range(NBUF):
            b = db
            b1 = (db + 1) % NBUF
            b2 = (db + 2) % NBUF
            ci = i + db

            gwait(b)

            @pl.when(ci + 1 < CHUNKS)
            def _():
                twait(b1)
                gstart(b1)

            scale(b)
            sstart(b)

            @pl.when(ci + 2 < CHUNKS)
            def _():
                @pl.when(ci >= 1)
                def _():
                    swait(b2)
                tstart(ci + 2, b2)

    swait((CHUNKS - 3) % NBUF)
    swait((CHUNKS - 2) % NBUF)
    swait((CHUNKS - 1) % NBUF)

    plsc.subcore_barrier()
    pltpu.sync_copy(acc_sh.at[rslice], out_hbm.at[c, rslice])


def _comb_body(p0_ref, p1_ref, b_ref, o_ref):
    o_ref[...] = p0_ref[0] + p1_ref[0] + b_ref[...]


def _combine(p, b):
    RB = 1000
    return pl.pallas_call(
        _comb_body,
        grid=(N_NODES // RB,),
        in_specs=[
            pl.BlockSpec((1, RB, D), lambda i: (0, i, 0)),
            pl.BlockSpec((1, RB, D), lambda i: (1, i, 0)),
            pl.BlockSpec((1, D), lambda i: (0, 0)),
        ],
        out_specs=pl.BlockSpec((RB, D), lambda i: (i, 0)),
        out_shape=jax.ShapeDtypeStruct((N_NODES, D), jnp.float32),
    )(p, p, b)


def kernel(x, rows, cols, values, W, b):
    support = _matmul(x, W)
    pad = E_PAD - N_EDGES
    rows1 = jnp.concatenate([rows, jnp.zeros((pad,), rows.dtype)])
    cols1 = jnp.concatenate([cols, jnp.zeros((pad,), cols.dtype)])
    vals1 = jnp.concatenate([values, jnp.zeros((pad,), values.dtype)])
    p = _sc_scatter(support, rows1, cols1, vals1)
    return _combine(p, b)
